# baseline (speedup 1.0000x reference)
import functools

import jax
import jax.numpy as jnp
from jax import lax
from jax.experimental import pallas as pl
from jax.experimental.pallas import tpu as pltpu
from jax.experimental.pallas import tpu_sc as plsc

_DIM = 128
_N = 100000
_NW = 32
_T = 128
_NT = 25
_NB = 6
_CHUNK = _T * _NT
_LASTB = _N - _CHUNK

_mesh = plsc.VectorSubcoreMesh(core_axis_name="c", subcore_axis_name="s")


@functools.partial(
    pl.kernel,
    out_type=jax.ShapeDtypeStruct((_N, _DIM), jnp.float32),
    mesh=_mesh,
    scratch_types=[
        pltpu.VMEM((_CHUNK,), jnp.int32),
        pltpu.VMEM((_NB, _T, _DIM), jnp.float32),
        pltpu.VMEM_SHARED((119, _DIM), jnp.float32),
        pltpu.SemaphoreType.DMA,
        pltpu.SemaphoreType.DMA,
    ],
)
def _sc_gather(idx_hbm, table_hbm, out_hbm, idx_v, rows_v, table_v, gsem,
               osem):
    sid = lax.axis_index("s")
    wid = sid * 2 + lax.axis_index("c")
    base = jnp.minimum(wid * _CHUNK, _LASTB)

    @pl.when(sid == 0)
    def _():
        pltpu.sync_copy(table_hbm, table_v)

    pltpu.sync_copy(idx_hbm.at[pl.ds(base, _CHUNK)], idx_v)
    plsc.subcore_barrier()

    def start_gather(t, slot):
        pltpu.async_copy(table_v.at[idx_v.at[pl.ds(t * _T, _T)]],
                         rows_v.at[slot], gsem)

    def wait_gather(slot):
        pltpu.make_async_copy(table_v.at[idx_v.at[pl.ds(0, _T)]],
                              rows_v.at[slot], gsem).wait()

    def start_store(t, slot):
        pltpu.async_copy(rows_v.at[slot],
                         out_hbm.at[pl.ds(base + t * _T, _T)], osem)

    def wait_store(t, slot):
        pltpu.make_async_copy(rows_v.at[slot],
                              out_hbm.at[pl.ds(base + t * _T, _T)],
                              osem).wait()

    for b in range(_NB):
        start_gather(b, b)
        if b >= 2:
            wait_gather(b - 2)
            start_store(b - 2, b - 2)

    def outer(p, carry):
        for b in range(_NB):
            t = p * _NB + b
            wait_store(t - _NB, b)
            start_gather(t, b)
            wait_gather((b - 2) % _NB)
            start_store(t - 2, (b - 2) % _NB)
        return carry

    lax.fori_loop(1, 4, outer, 0, unroll=False)

    wait_store(18, 0)
    start_gather(24, 0)
    wait_gather(4)
    start_store(22, 4)
    wait_gather(5)
    start_store(23, 5)
    wait_gather(0)
    start_store(24, 0)
    for _ in range(_NB):
        pltpu.make_async_copy(rows_v.at[0], out_hbm.at[pl.ds(0, _T)],
                              osem).wait()


def kernel(atom_features, state_features, embedding_table):
    atom_embeds = _sc_gather(atom_features, embedding_table)
    return (atom_embeds, state_features)


# speedup vs baseline: 1.0030x; 1.0030x over previous
import functools

import jax
import jax.numpy as jnp
from jax import lax
from jax.experimental import pallas as pl
from jax.experimental.pallas import tpu as pltpu
from jax.experimental.pallas import tpu_sc as plsc

_DIM = 128
_N = 100000
_NW = 32
_T = 128
_NT = 25
_NB = 5
_CHUNK = _T * _NT
_LASTB = _N - _CHUNK

_mesh = plsc.VectorSubcoreMesh(core_axis_name="c", subcore_axis_name="s")


@functools.partial(
    pl.kernel,
    out_type=jax.ShapeDtypeStruct((_N, _DIM), jnp.float32),
    mesh=_mesh,
    scratch_types=[
        pltpu.VMEM((_CHUNK,), jnp.int32),
        pltpu.VMEM((_NB, _T, _DIM), jnp.float32),
        pltpu.VMEM_SHARED((119, _DIM), jnp.float32),
        pltpu.SemaphoreType.DMA,
        pltpu.SemaphoreType.DMA,
    ],
)
def _sc_gather(idx_hbm, table_hbm, out_hbm, idx_v, rows_v, table_v, gsem,
               osem):
    sid = lax.axis_index("s")
    wid = sid * 2 + lax.axis_index("c")
    base = jnp.minimum(wid * _CHUNK, _LASTB)

    @pl.when(sid == 0)
    def _():
        pltpu.sync_copy(table_hbm, table_v)

    pltpu.sync_copy(idx_hbm.at[pl.ds(base, _CHUNK)], idx_v)
    plsc.subcore_barrier()

    def start_gather(t, slot):
        pltpu.async_copy(table_v.at[idx_v.at[pl.ds(t * _T, _T)]],
                         rows_v.at[slot], gsem)

    def wait_gather(slot):
        pltpu.make_async_copy(table_v.at[idx_v.at[pl.ds(0, _T)]],
                              rows_v.at[slot], gsem).wait()

    def start_store(t, slot):
        pltpu.async_copy(rows_v.at[slot],
                         out_hbm.at[pl.ds(base + t * _T, _T)], osem)

    def wait_store(t, slot):
        pltpu.make_async_copy(rows_v.at[slot],
                              out_hbm.at[pl.ds(base + t * _T, _T)],
                              osem).wait()

    for b in range(_NB):
        start_gather(b, b)
        if b >= 1:
            wait_gather(b - 1)
            start_store(b - 1, b - 1)

    def outer(p, carry):
        for b in range(_NB):
            t = p * _NB + b
            wait_store(t - _NB, b)
            start_gather(t, b)
            wait_gather((b - 1) % _NB)
            start_store(t - 1, (b - 1) % _NB)
        return carry

    lax.fori_loop(1, _NT // _NB, outer, 0, unroll=False)

    wait_gather(4)
    start_store(24, 4)
    for _ in range(_NB):
        pltpu.make_async_copy(rows_v.at[0], out_hbm.at[pl.ds(0, _T)],
                              osem).wait()


def kernel(atom_features, state_features, embedding_table):
    atom_embeds = _sc_gather(atom_features, embedding_table)
    return (atom_embeds, state_features)


# speedup vs baseline: 1.0207x; 1.0177x over previous
import functools

import jax
import jax.numpy as jnp
from jax import lax
from jax.experimental import pallas as pl
from jax.experimental.pallas import tpu as pltpu
from jax.experimental.pallas import tpu_sc as plsc

_DIM = 128
_N = 100000
_NW = 32
_T = 128
_NT = 25
_NB = 5
_CHUNK = _T * _NT
_LASTB = _N - _CHUNK

_mesh = plsc.VectorSubcoreMesh(core_axis_name="c", subcore_axis_name="s")


@functools.partial(
    pl.kernel,
    out_type=jax.ShapeDtypeStruct((_N, _DIM), jnp.float32),
    mesh=_mesh,
    scratch_types=[
        pltpu.VMEM((_CHUNK,), jnp.int32),
        pltpu.VMEM((_NB, _T, _DIM), jnp.float32),
        pltpu.VMEM_SHARED((119, _DIM), jnp.float32),
        pltpu.SemaphoreType.DMA,
        pltpu.SemaphoreType.DMA,
        pltpu.SemaphoreType.DMA,
    ],
)
def _sc_gather(idx_hbm, table_hbm, out_hbm, idx_v, rows_v, table_v, gsem,
               osem, ssem):
    sid = lax.axis_index("s")
    wid = sid * 2 + lax.axis_index("c")
    base = jnp.minimum(wid * _CHUNK, _LASTB)

    @pl.when(sid == 0)
    def _():
        pltpu.async_copy(table_hbm, table_v, ssem)

    pltpu.sync_copy(idx_hbm.at[pl.ds(base, _CHUNK)], idx_v)

    @pl.when(sid == 0)
    def _():
        pltpu.make_async_copy(table_hbm, table_v, ssem).wait()

    plsc.subcore_barrier()

    def start_gather(t, slot):
        pltpu.async_copy(table_v.at[idx_v.at[pl.ds(t * _T, _T)]],
                         rows_v.at[slot], gsem)

    def wait_gather(slot):
        pltpu.make_async_copy(table_v.at[idx_v.at[pl.ds(0, _T)]],
                              rows_v.at[slot], gsem).wait()

    def start_store(t, slot):
        pltpu.async_copy(rows_v.at[slot],
                         out_hbm.at[pl.ds(base + t * _T, _T)], osem)

    def wait_store(t, slot):
        pltpu.make_async_copy(rows_v.at[slot],
                              out_hbm.at[pl.ds(base + t * _T, _T)],
                              osem).wait()

    for b in range(_NB):
        start_gather(b, b)
        if b >= 1:
            wait_gather(b - 1)
            start_store(b - 1, b - 1)

    def outer(p, carry):
        for b in range(_NB):
            t = p * _NB + b
            wait_store(t - _NB, b)
            start_gather(t, b)
            wait_gather((b - 1) % _NB)
            start_store(t - 1, (b - 1) % _NB)
        return carry

    lax.fori_loop(1, _NT // _NB, outer, 0, unroll=False)

    wait_gather(4)
    start_store(24, 4)
    for _ in range(_NB):
        pltpu.make_async_copy(rows_v.at[0], out_hbm.at[pl.ds(0, _T)],
                              osem).wait()


def kernel(atom_features, state_features, embedding_table):
    atom_embeds = _sc_gather(atom_features, embedding_table)
    return (atom_embeds, state_features)
